# baseline (speedup 1.0000x reference)
import functools

import jax
import jax.numpy as jnp
from jax import lax
from jax.experimental import pallas as pl
from jax.experimental.pallas import tpu as pltpu
from jax.experimental.pallas import tpu_sc as plsc

N = 10000
D = 32
F_IN = 128
HID = 256
NCLS = 16

NP = 10240
BLK = 512
NB = NP // BLK
CHUNK = 128
NSC = 2
NTILE = 16
NW = NSC * NTILE


NBUF = 4


def _sc_gather(table, idx3, feat, dtype):
    _, n_chunks, chunk = idx3.shape
    per_w = n_chunks * chunk
    rows_total = NW * per_w
    n_iters = n_chunks // NBUF
    mesh = plsc.VectorSubcoreMesh(core_axis_name="c", subcore_axis_name="s")

    @functools.partial(
        pl.kernel,
        mesh=mesh,
        out_type=jax.ShapeDtypeStruct((rows_total, feat), dtype),
        scratch_types=[
            pltpu.VMEM((n_chunks, chunk), jnp.int32),
        ]
        + [pltpu.VMEM((chunk, feat), dtype) for _ in range(NBUF)]
        + [pltpu.SemaphoreType.DMA for _ in range(2 * NBUF)],
    )
    def gk(table_hbm, idx_hbm, out_hbm, idx_v, *rest):
        bufs = rest[:NBUF]
        gsems = rest[NBUF:2 * NBUF]
        osems = rest[2 * NBUF:]
        wid = lax.axis_index("s") * NSC + lax.axis_index("c")
        base = wid * per_w
        pltpu.sync_copy(idx_hbm.at[wid], idx_v)
        for b in range(NBUF):
            pltpu.async_copy(table_hbm.at[idx_v.at[b]], bufs[b], gsems[b])

        def body(k, carry):
            for b in range(NBUF):
                j = k * NBUF + b
                pltpu.make_async_copy(
                    table_hbm.at[idx_v.at[j]], bufs[b], gsems[b]).wait()
                out_slice = out_hbm.at[pl.ds(base + j * chunk, chunk)]
                pltpu.async_copy(bufs[b], out_slice, osems[b])

                @pl.when(k < n_iters - 1)
                def _():
                    pltpu.make_async_copy(bufs[b], out_slice, osems[b]).wait()
                    pltpu.async_copy(
                        table_hbm.at[idx_v.at[j + NBUF]], bufs[b], gsems[b])
            return carry

        lax.fori_loop(0, n_iters, body, 0)
        for b in range(NBUF):
            j = (n_iters - 1) * NBUF + b
            out_slice = out_hbm.at[pl.ds(base + j * chunk, chunk)]
            pltpu.make_async_copy(bufs[b], out_slice, osems[b]).wait()

    return gk(table, idx3)


def _conv1(m, xp, W_ihT, W_hhT, bias, W_self1, W_neigh1, b1, W_pool, b_pool):
    def body(m_ref, x_ref, wih_ref, whh_ref, b_ref, ws_ref, wn_ref, b1_ref,
             wp_ref, bp_ref, out_ref, q_ref, h_s, c_s):
        d = pl.program_id(1)

        @pl.when(d == 0)
        def _():
            h_s[...] = jnp.zeros_like(h_s)
            c_s[...] = jnp.zeros_like(c_s)

        xt = m_ref[0].astype(jnp.bfloat16)
        gates = (jnp.dot(xt, wih_ref[...], preferred_element_type=jnp.float32)
                 + jnp.dot(h_s[...], whh_ref[...], preferred_element_type=jnp.float32)
                 + b_ref[...])
        gi = jax.nn.sigmoid(gates[:, 0:F_IN])
        gf = jax.nn.sigmoid(gates[:, F_IN:2 * F_IN])
        gg = jnp.tanh(gates[:, 2 * F_IN:3 * F_IN])
        go = jax.nn.sigmoid(gates[:, 3 * F_IN:4 * F_IN])
        c = gf * c_s[...] + gi * gg
        h = go * jnp.tanh(c)
        c_s[...] = c
        h_s[...] = h.astype(jnp.bfloat16)

        @pl.when(d == D - 1)
        def _():
            hh = jax.nn.relu(
                jnp.dot(x_ref[...], ws_ref[...], preferred_element_type=jnp.float32)
                + jnp.dot(h.astype(jnp.bfloat16), wn_ref[...],
                          preferred_element_type=jnp.float32)
                + b1_ref[...])
            out_ref[...] = hh
            q_ref[...] = jax.nn.relu(
                jnp.dot(hh.astype(jnp.bfloat16), wp_ref[...],
                        preferred_element_type=jnp.float32)
                + bp_ref[...]).astype(jnp.bfloat16)

    return pl.pallas_call(
        body,
        grid=(NB, D),
        in_specs=[
            pl.BlockSpec((1, BLK, F_IN), lambda i, d: (d, i, 0)),
            pl.BlockSpec((BLK, F_IN), lambda i, d: (i, 0)),
            pl.BlockSpec((F_IN, 4 * F_IN), lambda i, d: (0, 0)),
            pl.BlockSpec((F_IN, 4 * F_IN), lambda i, d: (0, 0)),
            pl.BlockSpec((1, 4 * F_IN), lambda i, d: (0, 0)),
            pl.BlockSpec((F_IN, HID), lambda i, d: (0, 0)),
            pl.BlockSpec((F_IN, HID), lambda i, d: (0, 0)),
            pl.BlockSpec((1, HID), lambda i, d: (0, 0)),
            pl.BlockSpec((HID, HID), lambda i, d: (0, 0)),
            pl.BlockSpec((1, HID), lambda i, d: (0, 0)),
        ],
        out_specs=[
            pl.BlockSpec((BLK, HID), lambda i, d: (i, 0)),
            pl.BlockSpec((BLK, HID), lambda i, d: (i, 0)),
        ],
        out_shape=[
            jax.ShapeDtypeStruct((NP, HID), jnp.float32),
            jax.ShapeDtypeStruct((NP, HID), jnp.bfloat16),
        ],
        scratch_shapes=[
            pltpu.VMEM((BLK, F_IN), jnp.bfloat16),
            pltpu.VMEM((BLK, F_IN), jnp.float32),
        ],
    )(m, xp, W_ihT, W_hhT, bias, W_self1, W_neigh1, b1, W_pool, b_pool)


def _conv2(m2, h, W_self2, W_neigh2, b2):
    def body(m_ref, h_ref, ws_ref, wn_ref, b2_ref, out_ref, mx_s):
        d = pl.program_id(1)
        t = m_ref[0]
        prev = jnp.where(d == 0, jnp.zeros_like(t), mx_s[...])
        mx = jnp.maximum(t, prev)
        mx_s[...] = mx

        @pl.when(d == D - 1)
        def _():
            out_ref[...] = (
                jnp.dot(h_ref[...].astype(jnp.bfloat16), ws_ref[...],
                        preferred_element_type=jnp.float32)
                + jnp.dot(mx, wn_ref[...], preferred_element_type=jnp.float32)
                + b2_ref[...])

    return pl.pallas_call(
        body,
        grid=(NB, D),
        in_specs=[
            pl.BlockSpec((1, BLK, HID), lambda i, d: (d, i, 0)),
            pl.BlockSpec((BLK, HID), lambda i, d: (i, 0)),
            pl.BlockSpec((HID, NCLS), lambda i, d: (0, 0)),
            pl.BlockSpec((HID, NCLS), lambda i, d: (0, 0)),
            pl.BlockSpec((1, NCLS), lambda i, d: (0, 0)),
        ],
        out_specs=pl.BlockSpec((BLK, NCLS), lambda i, d: (i, 0)),
        out_shape=jax.ShapeDtypeStruct((NP, NCLS), jnp.float32),
        scratch_shapes=[pltpu.VMEM((BLK, HID), jnp.float32)],
    )(m2, h, W_self2, W_neigh2, b2)


def kernel(x, nbr, W_ih, W_hh, b_ih, b_hh, W_self1, W_neigh1, b1,
           W_pool, b_pool, W_self2, W_neigh2, b2):
    bf = jnp.bfloat16
    nbr = nbr.astype(jnp.int32)
    xp = jnp.pad(x.astype(bf), ((0, NP - N), (0, 0)))
    idx = jnp.pad(nbr.T, ((0, 0), (0, NP - N))).reshape(-1)
    per_w = (D * NP) // NW
    idx3 = idx.reshape(NW, per_w // CHUNK, CHUNK)

    m = _sc_gather(x, idx3, F_IN, jnp.float32).reshape(D, NP, F_IN)
    bias = (b_ih + b_hh).reshape(1, 4 * F_IN)
    h, q = _conv1(m, xp, W_ih.T.astype(bf), W_hh.T.astype(bf), bias,
                  W_self1.astype(bf), W_neigh1.astype(bf), b1.reshape(1, HID),
                  W_pool.astype(bf), b_pool.reshape(1, HID))
    q_i32 = lax.bitcast_convert_type(q.reshape(NP, HID // 2, 2), jnp.int32)
    m2 = lax.bitcast_convert_type(
        _sc_gather(q_i32, idx3, HID // 2, jnp.int32), bf).reshape(D, NP, HID)
    out = _conv2(m2, h, W_self2.astype(bf), W_neigh2.astype(bf),
                 b2.reshape(1, NCLS))
    return out[:N]


# speedup vs baseline: 1.7286x; 1.7286x over previous
import functools

import jax
import jax.numpy as jnp
from jax import lax
from jax.experimental import pallas as pl
from jax.experimental.pallas import tpu as pltpu
from jax.experimental.pallas import tpu_sc as plsc

N = 10000
D = 32
F_IN = 128
HID = 256
NCLS = 16

NP = 10240
BLK = 512
NB = NP // BLK
CHUNK = 128
NSC = 2
NTILE = 16
NW = NSC * NTILE


NBUF = 4


def _sc_gather(table, idx3, feat, dtype):
    _, n_chunks, chunk = idx3.shape
    per_w = n_chunks * chunk
    rows_total = NW * per_w
    n_iters = n_chunks // NBUF
    mesh = plsc.VectorSubcoreMesh(core_axis_name="c", subcore_axis_name="s")

    @functools.partial(
        pl.kernel,
        mesh=mesh,
        out_type=jax.ShapeDtypeStruct((rows_total, feat), dtype),
        scratch_types=[
            pltpu.VMEM((n_chunks, chunk), jnp.int32),
        ]
        + [pltpu.VMEM((chunk, feat), dtype) for _ in range(NBUF)]
        + [pltpu.SemaphoreType.DMA for _ in range(2 * NBUF)],
    )
    def gk(table_hbm, idx_hbm, out_hbm, idx_v, *rest):
        bufs = rest[:NBUF]
        gsems = rest[NBUF:2 * NBUF]
        osems = rest[2 * NBUF:]
        wid = lax.axis_index("s") * NSC + lax.axis_index("c")
        base = wid * per_w
        pltpu.sync_copy(idx_hbm.at[wid], idx_v)
        for b in range(NBUF):
            pltpu.async_copy(table_hbm.at[idx_v.at[b]], bufs[b], gsems[b])

        def body(k, carry):
            for b in range(NBUF):
                j = k * NBUF + b
                pltpu.make_async_copy(
                    table_hbm.at[idx_v.at[j]], bufs[b], gsems[b]).wait()
                out_slice = out_hbm.at[pl.ds(base + j * chunk, chunk)]
                pltpu.async_copy(bufs[b], out_slice, osems[b])

                @pl.when(k < n_iters - 1)
                def _():
                    pltpu.make_async_copy(bufs[b], out_slice, osems[b]).wait()
                    pltpu.async_copy(
                        table_hbm.at[idx_v.at[j + NBUF]], bufs[b], gsems[b])
            return carry

        lax.fori_loop(0, n_iters, body, 0)
        for b in range(NBUF):
            j = (n_iters - 1) * NBUF + b
            out_slice = out_hbm.at[pl.ds(base + j * chunk, chunk)]
            pltpu.make_async_copy(bufs[b], out_slice, osems[b]).wait()

    return gk(table, idx3)


def _conv1(m, xp, W_ihT, W_hhT, bias, W_self1, W_neigh1, b1, W_pool, b_pool):
    def body(m_ref, x_ref, wih_ref, whh_ref, b_ref, ws_ref, wn_ref, b1_ref,
             wp_ref, bp_ref, out_ref, q_ref, h_s, c_s):
        d = pl.program_id(1)

        @pl.when(d == 0)
        def _():
            h_s[...] = jnp.zeros_like(h_s)
            c_s[...] = jnp.zeros_like(c_s)

        xt = m_ref[0].astype(jnp.bfloat16)
        gates = (jnp.dot(xt, wih_ref[...], preferred_element_type=jnp.float32)
                 + jnp.dot(h_s[...], whh_ref[...], preferred_element_type=jnp.float32)
                 + b_ref[...])
        gi = jax.nn.sigmoid(gates[:, 0:F_IN])
        gf = jax.nn.sigmoid(gates[:, F_IN:2 * F_IN])
        gg = jnp.tanh(gates[:, 2 * F_IN:3 * F_IN])
        go = jax.nn.sigmoid(gates[:, 3 * F_IN:4 * F_IN])
        c = gf * c_s[...] + gi * gg
        h = go * jnp.tanh(c)
        c_s[...] = c
        h_s[...] = h.astype(jnp.bfloat16)

        @pl.when(d == D - 1)
        def _():
            hh = jax.nn.relu(
                jnp.dot(x_ref[...], ws_ref[...], preferred_element_type=jnp.float32)
                + jnp.dot(h.astype(jnp.bfloat16), wn_ref[...],
                          preferred_element_type=jnp.float32)
                + b1_ref[...])
            out_ref[...] = hh
            q_ref[...] = jax.nn.relu(
                jnp.dot(hh.astype(jnp.bfloat16), wp_ref[...],
                        preferred_element_type=jnp.float32)
                + bp_ref[...])

    return pl.pallas_call(
        body,
        grid=(NB, D),
        in_specs=[
            pl.BlockSpec((1, BLK, F_IN), lambda i, d: (d, i, 0)),
            pl.BlockSpec((BLK, F_IN), lambda i, d: (i, 0)),
            pl.BlockSpec((F_IN, 4 * F_IN), lambda i, d: (0, 0)),
            pl.BlockSpec((F_IN, 4 * F_IN), lambda i, d: (0, 0)),
            pl.BlockSpec((1, 4 * F_IN), lambda i, d: (0, 0)),
            pl.BlockSpec((F_IN, HID), lambda i, d: (0, 0)),
            pl.BlockSpec((F_IN, HID), lambda i, d: (0, 0)),
            pl.BlockSpec((1, HID), lambda i, d: (0, 0)),
            pl.BlockSpec((HID, HID), lambda i, d: (0, 0)),
            pl.BlockSpec((1, HID), lambda i, d: (0, 0)),
        ],
        out_specs=[
            pl.BlockSpec((BLK, HID), lambda i, d: (i, 0)),
            pl.BlockSpec((BLK, HID), lambda i, d: (i, 0)),
        ],
        out_shape=[
            jax.ShapeDtypeStruct((NP, HID), jnp.float32),
            jax.ShapeDtypeStruct((NP, HID), jnp.float32),
        ],
        scratch_shapes=[
            pltpu.VMEM((BLK, F_IN), jnp.bfloat16),
            pltpu.VMEM((BLK, F_IN), jnp.float32),
        ],
    )(m, xp, W_ihT, W_hhT, bias, W_self1, W_neigh1, b1, W_pool, b_pool)


def _conv2(m2, h, W_self2, W_neigh2, b2):
    def body(m_ref, h_ref, ws_ref, wn_ref, b2_ref, out_ref, mx_s):
        d = pl.program_id(1)
        t = m_ref[0]
        prev = jnp.where(d == 0, jnp.zeros_like(t), mx_s[...])
        mx = jnp.maximum(t, prev)
        mx_s[...] = mx

        @pl.when(d == D - 1)
        def _():
            out_ref[...] = (
                jnp.dot(h_ref[...].astype(jnp.bfloat16), ws_ref[...],
                        preferred_element_type=jnp.float32)
                + jnp.dot(mx.astype(jnp.bfloat16), wn_ref[...],
                          preferred_element_type=jnp.float32)
                + b2_ref[...])

    return pl.pallas_call(
        body,
        grid=(NB, D),
        in_specs=[
            pl.BlockSpec((1, BLK, HID), lambda i, d: (d, i, 0)),
            pl.BlockSpec((BLK, HID), lambda i, d: (i, 0)),
            pl.BlockSpec((HID, NCLS), lambda i, d: (0, 0)),
            pl.BlockSpec((HID, NCLS), lambda i, d: (0, 0)),
            pl.BlockSpec((1, NCLS), lambda i, d: (0, 0)),
        ],
        out_specs=pl.BlockSpec((BLK, NCLS), lambda i, d: (i, 0)),
        out_shape=jax.ShapeDtypeStruct((NP, NCLS), jnp.float32),
        scratch_shapes=[pltpu.VMEM((BLK, HID), jnp.float32)],
    )(m2, h, W_self2, W_neigh2, b2)


def kernel(x, nbr, W_ih, W_hh, b_ih, b_hh, W_self1, W_neigh1, b1,
           W_pool, b_pool, W_self2, W_neigh2, b2):
    bf = jnp.bfloat16
    nbr = nbr.astype(jnp.int32)
    xp = jnp.pad(x.astype(bf), ((0, NP - N), (0, 0)))
    idx = jnp.pad(nbr.T, ((0, 0), (0, NP - N))).reshape(-1)
    per_w = (D * NP) // NW
    c1 = 16384 // F_IN
    c2 = 16384 // HID
    idx3_1 = idx.reshape(NW, per_w // c1, c1)
    idx3_2 = idx.reshape(NW, per_w // c2, c2)

    m = _sc_gather(x, idx3_1, F_IN, jnp.float32).reshape(D, NP, F_IN)
    bias = (b_ih + b_hh).reshape(1, 4 * F_IN)
    h, q = _conv1(m, xp, W_ih.T.astype(bf), W_hh.T.astype(bf), bias,
                  W_self1.astype(bf), W_neigh1.astype(bf), b1.reshape(1, HID),
                  W_pool.astype(bf), b_pool.reshape(1, HID))
    m2 = _sc_gather(q, idx3_2, HID, jnp.float32).reshape(D, NP, HID)
    out = _conv2(m2, h, W_self2.astype(bf), W_neigh2.astype(bf),
                 b2.reshape(1, NCLS))
    return out[:N]


# speedup vs baseline: 2.2050x; 1.2756x over previous
import functools

import jax
import jax.numpy as jnp
from jax import lax
from jax.experimental import pallas as pl
from jax.experimental.pallas import tpu as pltpu
from jax.experimental.pallas import tpu_sc as plsc

N = 10000
D = 32
F_IN = 128
HID = 256
NCLS = 16

NP = 10240
BLK = 512
NB = NP // BLK
CHUNK = 128
NSC = 2
NTILE = 16
NW = NSC * NTILE


NBUF = 4


def _sc_gather(table, idx3, feat, dtype):
    _, n_chunks, chunk = idx3.shape
    per_w = n_chunks * chunk
    rows_total = NW * per_w
    n_iters = n_chunks // NBUF
    mesh = plsc.VectorSubcoreMesh(core_axis_name="c", subcore_axis_name="s")

    @functools.partial(
        pl.kernel,
        mesh=mesh,
        out_type=jax.ShapeDtypeStruct((rows_total, feat), dtype),
        scratch_types=[
            pltpu.VMEM((n_chunks, chunk), jnp.int32),
        ]
        + [pltpu.VMEM((chunk, feat), dtype) for _ in range(NBUF)]
        + [pltpu.SemaphoreType.DMA for _ in range(2 * NBUF)],
    )
    def gk(table_hbm, idx_hbm, out_hbm, idx_v, *rest):
        bufs = rest[:NBUF]
        gsems = rest[NBUF:2 * NBUF]
        osems = rest[2 * NBUF:]
        wid = lax.axis_index("s") * NSC + lax.axis_index("c")
        base = wid * per_w
        pltpu.sync_copy(idx_hbm.at[wid], idx_v)
        for b in range(NBUF):
            pltpu.async_copy(table_hbm.at[idx_v.at[b]], bufs[b], gsems[b])

        def body(k, carry):
            for b in range(NBUF):
                j = k * NBUF + b
                pltpu.make_async_copy(
                    table_hbm.at[idx_v.at[j]], bufs[b], gsems[b]).wait()
                out_slice = out_hbm.at[pl.ds(base + j * chunk, chunk)]
                pltpu.async_copy(bufs[b], out_slice, osems[b])

                @pl.when(k < n_iters - 1)
                def _():
                    pltpu.make_async_copy(bufs[b], out_slice, osems[b]).wait()
                    pltpu.async_copy(
                        table_hbm.at[idx_v.at[j + NBUF]], bufs[b], gsems[b])
            return carry

        lax.fori_loop(0, n_iters, body, 0)
        for b in range(NBUF):
            j = (n_iters - 1) * NBUF + b
            out_slice = out_hbm.at[pl.ds(base + j * chunk, chunk)]
            pltpu.make_async_copy(bufs[b], out_slice, osems[b]).wait()

    return gk(table, idx3)


def _conv1(m, xp, W_ihT, W_hhT, bias, W_self1, W_neigh1, b1, W_pool, b_pool):
    def body(m_ref, x_ref, wih_ref, whh_ref, b_ref, ws_ref, wn_ref, b1_ref,
             wp_ref, bp_ref, out_ref, q_ref, h_s, c_s):
        d = pl.program_id(1)

        @pl.when(d == 0)
        def _():
            h_s[...] = jnp.zeros_like(h_s)
            c_s[...] = jnp.zeros_like(c_s)

        xt = m_ref[0].astype(jnp.bfloat16)
        gates = (jnp.dot(xt, wih_ref[...], preferred_element_type=jnp.float32)
                 + jnp.dot(h_s[...], whh_ref[...], preferred_element_type=jnp.float32)
                 + b_ref[...])
        gi = jax.nn.sigmoid(gates[:, 0:F_IN])
        gf = jax.nn.sigmoid(gates[:, F_IN:2 * F_IN])
        gg = jnp.tanh(gates[:, 2 * F_IN:3 * F_IN])
        go = jax.nn.sigmoid(gates[:, 3 * F_IN:4 * F_IN])
        c = gf * c_s[...] + gi * gg
        h = go * jnp.tanh(c)
        c_s[...] = c
        h_s[...] = h.astype(jnp.bfloat16)

        @pl.when(d == D - 1)
        def _():
            hh = jax.nn.relu(
                jnp.dot(x_ref[...], ws_ref[...], preferred_element_type=jnp.float32)
                + jnp.dot(h.astype(jnp.bfloat16), wn_ref[...],
                          preferred_element_type=jnp.float32)
                + b1_ref[...])
            out_ref[...] = hh
            q_ref[...] = jax.nn.relu(
                jnp.dot(hh.astype(jnp.bfloat16), wp_ref[...],
                        preferred_element_type=jnp.float32)
                + bp_ref[...])

    cn = m.shape[1]
    return pl.pallas_call(
        body,
        grid=(cn // BLK, D),
        in_specs=[
            pl.BlockSpec((1, BLK, F_IN), lambda i, d: (d, i, 0)),
            pl.BlockSpec((BLK, F_IN), lambda i, d: (i, 0)),
            pl.BlockSpec((F_IN, 4 * F_IN), lambda i, d: (0, 0)),
            pl.BlockSpec((F_IN, 4 * F_IN), lambda i, d: (0, 0)),
            pl.BlockSpec((1, 4 * F_IN), lambda i, d: (0, 0)),
            pl.BlockSpec((F_IN, HID), lambda i, d: (0, 0)),
            pl.BlockSpec((F_IN, HID), lambda i, d: (0, 0)),
            pl.BlockSpec((1, HID), lambda i, d: (0, 0)),
            pl.BlockSpec((HID, HID), lambda i, d: (0, 0)),
            pl.BlockSpec((1, HID), lambda i, d: (0, 0)),
        ],
        out_specs=[
            pl.BlockSpec((BLK, HID), lambda i, d: (i, 0)),
            pl.BlockSpec((BLK, HID), lambda i, d: (i, 0)),
        ],
        out_shape=[
            jax.ShapeDtypeStruct((cn, HID), jnp.float32),
            jax.ShapeDtypeStruct((cn, HID), jnp.float32),
        ],
        scratch_shapes=[
            pltpu.VMEM((BLK, F_IN), jnp.bfloat16),
            pltpu.VMEM((BLK, F_IN), jnp.float32),
        ],
    )(m, xp, W_ihT, W_hhT, bias, W_self1, W_neigh1, b1, W_pool, b_pool)


def _conv2(m2, h, W_self2, W_neigh2, b2):
    def body(m_ref, h_ref, ws_ref, wn_ref, b2_ref, out_ref, mx_s):
        d = pl.program_id(1)
        t = m_ref[0]
        prev = jnp.where(d == 0, jnp.zeros_like(t), mx_s[...])
        mx = jnp.maximum(t, prev)
        mx_s[...] = mx

        @pl.when(d == D - 1)
        def _():
            out_ref[...] = (
                jnp.dot(h_ref[...].astype(jnp.bfloat16), ws_ref[...],
                        preferred_element_type=jnp.float32)
                + jnp.dot(mx.astype(jnp.bfloat16), wn_ref[...],
                          preferred_element_type=jnp.float32)
                + b2_ref[...])

    cn = m2.shape[1]
    return pl.pallas_call(
        body,
        grid=(cn // BLK, D),
        in_specs=[
            pl.BlockSpec((1, BLK, HID), lambda i, d: (d, i, 0)),
            pl.BlockSpec((BLK, HID), lambda i, d: (i, 0)),
            pl.BlockSpec((HID, NCLS), lambda i, d: (0, 0)),
            pl.BlockSpec((HID, NCLS), lambda i, d: (0, 0)),
            pl.BlockSpec((1, NCLS), lambda i, d: (0, 0)),
        ],
        out_specs=pl.BlockSpec((BLK, NCLS), lambda i, d: (i, 0)),
        out_shape=jax.ShapeDtypeStruct((cn, NCLS), jnp.float32),
        scratch_shapes=[pltpu.VMEM((BLK, HID), jnp.float32)],
    )(m2, h, W_self2, W_neigh2, b2)


NCHUNK = 4
CN = NP // NCHUNK


def kernel(x, nbr, W_ih, W_hh, b_ih, b_hh, W_self1, W_neigh1, b1,
           W_pool, b_pool, W_self2, W_neigh2, b2):
    bf = jnp.bfloat16
    nbr = nbr.astype(jnp.int32)
    xp = jnp.pad(x.astype(bf), ((0, NP - N), (0, 0)))
    idxt = jnp.pad(nbr.T, ((0, 0), (0, NP - N)))
    per_w = (D * CN) // NW
    c1 = 16384 // F_IN
    c2 = 16384 // HID
    idx_k = [idxt[:, k * CN:(k + 1) * CN].reshape(-1) for k in range(NCHUNK)]

    bias = (b_ih + b_hh).reshape(1, 4 * F_IN)
    wih, whh = W_ih.T.astype(bf), W_hh.T.astype(bf)
    ws1, wn1 = W_self1.astype(bf), W_neigh1.astype(bf)
    wp = W_pool.astype(bf)
    ws2, wn2 = W_self2.astype(bf), W_neigh2.astype(bf)

    hs, qs = [], []
    for k in range(NCHUNK):
        mk = _sc_gather(x, idx_k[k].reshape(NW, per_w // c1, c1), F_IN,
                        jnp.float32).reshape(D, CN, F_IN)
        hk, qk = _conv1(mk, xp[k * CN:(k + 1) * CN], wih, whh, bias,
                        ws1, wn1, b1.reshape(1, HID), wp,
                        b_pool.reshape(1, HID))
        hs.append(hk)
        qs.append(qk)
    q = jnp.concatenate(qs, axis=0)

    outs = []
    for k in range(NCHUNK):
        m2k = _sc_gather(q, idx_k[k].reshape(NW, per_w // c2, c2), HID,
                         jnp.float32).reshape(D, CN, HID)
        outs.append(_conv2(m2k, hs[k], ws2, wn2, b2.reshape(1, NCLS)))
    return jnp.concatenate(outs, axis=0)[:N]


# speedup vs baseline: 2.2247x; 1.0089x over previous
import functools

import jax
import jax.numpy as jnp
from jax import lax
from jax.experimental import pallas as pl
from jax.experimental.pallas import tpu as pltpu
from jax.experimental.pallas import tpu_sc as plsc

N = 10000
D = 32
F_IN = 128
HID = 256
NCLS = 16

NP = 10240
BLK = 512
NB = NP // BLK
CHUNK = 128
NSC = 2
NTILE = 16
NW = NSC * NTILE


NBUF = 4


def _sc_gather(table, idx3, feat, dtype):
    _, n_chunks, chunk = idx3.shape
    per_w = n_chunks * chunk
    rows_total = NW * per_w
    n_iters = n_chunks // NBUF
    mesh = plsc.VectorSubcoreMesh(core_axis_name="c", subcore_axis_name="s")

    @functools.partial(
        pl.kernel,
        mesh=mesh,
        out_type=jax.ShapeDtypeStruct((rows_total, feat), dtype),
        scratch_types=[
            pltpu.VMEM((n_chunks, chunk), jnp.int32),
        ]
        + [pltpu.VMEM((chunk, feat), dtype) for _ in range(NBUF)]
        + [pltpu.SemaphoreType.DMA for _ in range(2 * NBUF)],
    )
    def gk(table_hbm, idx_hbm, out_hbm, idx_v, *rest):
        bufs = rest[:NBUF]
        gsems = rest[NBUF:2 * NBUF]
        osems = rest[2 * NBUF:]
        wid = lax.axis_index("s") * NSC + lax.axis_index("c")
        base = wid * per_w
        pltpu.sync_copy(idx_hbm.at[wid], idx_v)
        for b in range(NBUF):
            pltpu.async_copy(table_hbm.at[idx_v.at[b]], bufs[b], gsems[b])

        def body(k, carry):
            for b in range(NBUF):
                j = k * NBUF + b
                pltpu.make_async_copy(
                    table_hbm.at[idx_v.at[j]], bufs[b], gsems[b]).wait()
                out_slice = out_hbm.at[pl.ds(base + j * chunk, chunk)]
                pltpu.async_copy(bufs[b], out_slice, osems[b])

                @pl.when(k < n_iters - 1)
                def _():
                    pltpu.make_async_copy(bufs[b], out_slice, osems[b]).wait()
                    pltpu.async_copy(
                        table_hbm.at[idx_v.at[j + NBUF]], bufs[b], gsems[b])
            return carry

        lax.fori_loop(0, n_iters, body, 0)
        for b in range(NBUF):
            j = (n_iters - 1) * NBUF + b
            out_slice = out_hbm.at[pl.ds(base + j * chunk, chunk)]
            pltpu.make_async_copy(bufs[b], out_slice, osems[b]).wait()

    return gk(table, idx3)


def _conv1(m, xp, W_ihT, W_hhT, bias, W_self1, W_neigh1, b1, W_pool, b_pool):
    def body(m_ref, x_ref, wih_ref, whh_ref, b_ref, ws_ref, wn_ref, b1_ref,
             wp_ref, bp_ref, out_ref, q_ref, h_s, c_s):
        d = pl.program_id(1)

        @pl.when(d == 0)
        def _():
            h_s[...] = jnp.zeros_like(h_s)
            c_s[...] = jnp.zeros_like(c_s)

        xt = m_ref[0].astype(jnp.bfloat16)
        hb = h_s[...]
        pre_if = (jnp.dot(xt, wih_ref[:, 0:2 * F_IN],
                          preferred_element_type=jnp.float32)
                  + jnp.dot(hb, whh_ref[:, 0:2 * F_IN],
                            preferred_element_type=jnp.float32)
                  + b_ref[:, 0:2 * F_IN])
        pre_go = (jnp.dot(xt, wih_ref[:, 2 * F_IN:4 * F_IN],
                          preferred_element_type=jnp.float32)
                  + jnp.dot(hb, whh_ref[:, 2 * F_IN:4 * F_IN],
                            preferred_element_type=jnp.float32)
                  + b_ref[:, 2 * F_IN:4 * F_IN])
        def sigm(v):
            return 0.5 * jnp.tanh(v) + 0.5

        gi = sigm(pre_if[:, 0:F_IN])
        gf = sigm(pre_if[:, F_IN:2 * F_IN])
        gg = jnp.tanh(pre_go[:, 0:F_IN])
        go = sigm(pre_go[:, F_IN:2 * F_IN])
        c = gf * c_s[...] + gi * gg
        h = go * jnp.tanh(c)
        c_s[...] = c
        h_s[...] = h.astype(jnp.bfloat16)

        @pl.when(d == D - 1)
        def _():
            hh = jax.nn.relu(
                jnp.dot(x_ref[...], ws_ref[...], preferred_element_type=jnp.float32)
                + jnp.dot(h.astype(jnp.bfloat16), wn_ref[...],
                          preferred_element_type=jnp.float32)
                + b1_ref[...])
            out_ref[...] = hh
            q_ref[...] = jax.nn.relu(
                jnp.dot(hh.astype(jnp.bfloat16), wp_ref[...],
                        preferred_element_type=jnp.float32)
                + bp_ref[...])

    cn = m.shape[1]
    return pl.pallas_call(
        body,
        grid=(cn // BLK, D),
        in_specs=[
            pl.BlockSpec((1, BLK, F_IN), lambda i, d: (d, i, 0)),
            pl.BlockSpec((BLK, F_IN), lambda i, d: (i, 0)),
            pl.BlockSpec((F_IN, 4 * F_IN), lambda i, d: (0, 0)),
            pl.BlockSpec((F_IN, 4 * F_IN), lambda i, d: (0, 0)),
            pl.BlockSpec((1, 4 * F_IN), lambda i, d: (0, 0)),
            pl.BlockSpec((F_IN, HID), lambda i, d: (0, 0)),
            pl.BlockSpec((F_IN, HID), lambda i, d: (0, 0)),
            pl.BlockSpec((1, HID), lambda i, d: (0, 0)),
            pl.BlockSpec((HID, HID), lambda i, d: (0, 0)),
            pl.BlockSpec((1, HID), lambda i, d: (0, 0)),
        ],
        out_specs=[
            pl.BlockSpec((BLK, HID), lambda i, d: (i, 0)),
            pl.BlockSpec((BLK, HID), lambda i, d: (i, 0)),
        ],
        out_shape=[
            jax.ShapeDtypeStruct((cn, HID), jnp.float32),
            jax.ShapeDtypeStruct((cn, HID), jnp.float32),
        ],
        scratch_shapes=[
            pltpu.VMEM((BLK, F_IN), jnp.bfloat16),
            pltpu.VMEM((BLK, F_IN), jnp.float32),
        ],
    )(m, xp, W_ihT, W_hhT, bias, W_self1, W_neigh1, b1, W_pool, b_pool)


def _conv2(m2, h, W_self2, W_neigh2, b2):
    def body(m_ref, h_ref, ws_ref, wn_ref, b2_ref, out_ref, mx_s):
        d = pl.program_id(1)
        t = m_ref[0]
        prev = jnp.where(d == 0, jnp.zeros_like(t), mx_s[...])
        mx = jnp.maximum(t, prev)
        mx_s[...] = mx

        @pl.when(d == D - 1)
        def _():
            out_ref[...] = (
                jnp.dot(h_ref[...].astype(jnp.bfloat16), ws_ref[...],
                        preferred_element_type=jnp.float32)
                + jnp.dot(mx.astype(jnp.bfloat16), wn_ref[...],
                          preferred_element_type=jnp.float32)
                + b2_ref[...])

    cn = m2.shape[1]
    return pl.pallas_call(
        body,
        grid=(cn // BLK, D),
        in_specs=[
            pl.BlockSpec((1, BLK, HID), lambda i, d: (d, i, 0)),
            pl.BlockSpec((BLK, HID), lambda i, d: (i, 0)),
            pl.BlockSpec((HID, NCLS), lambda i, d: (0, 0)),
            pl.BlockSpec((HID, NCLS), lambda i, d: (0, 0)),
            pl.BlockSpec((1, NCLS), lambda i, d: (0, 0)),
        ],
        out_specs=pl.BlockSpec((BLK, NCLS), lambda i, d: (i, 0)),
        out_shape=jax.ShapeDtypeStruct((cn, NCLS), jnp.float32),
        scratch_shapes=[pltpu.VMEM((BLK, HID), jnp.float32)],
    )(m2, h, W_self2, W_neigh2, b2)


NCHUNK = 4
CN = NP // NCHUNK


def kernel(x, nbr, W_ih, W_hh, b_ih, b_hh, W_self1, W_neigh1, b1,
           W_pool, b_pool, W_self2, W_neigh2, b2):
    bf = jnp.bfloat16
    nbr = nbr.astype(jnp.int32)
    xp = jnp.pad(x.astype(bf), ((0, NP - N), (0, 0)))
    idxt = jnp.pad(nbr.T, ((0, 0), (0, NP - N)))
    per_w = (D * CN) // NW
    c1 = 16384 // F_IN
    c2 = 16384 // HID
    idx_k = [idxt[:, k * CN:(k + 1) * CN].reshape(-1) for k in range(NCHUNK)]

    gate_scale = jnp.concatenate([
        jnp.full((2 * F_IN,), 0.5, jnp.float32),
        jnp.ones((F_IN,), jnp.float32),
        jnp.full((F_IN,), 0.5, jnp.float32),
    ])
    bias = ((b_ih + b_hh) * gate_scale).reshape(1, 4 * F_IN)
    wih = (W_ih.T * gate_scale[None, :]).astype(bf)
    whh = (W_hh.T * gate_scale[None, :]).astype(bf)
    ws1, wn1 = W_self1.astype(bf), W_neigh1.astype(bf)
    wp = W_pool.astype(bf)
    ws2, wn2 = W_self2.astype(bf), W_neigh2.astype(bf)

    hs, qs = [], []
    for k in range(NCHUNK):
        mk = _sc_gather(x, idx_k[k].reshape(NW, per_w // c1, c1), F_IN,
                        jnp.float32).reshape(D, CN, F_IN)
        hk, qk = _conv1(mk, xp[k * CN:(k + 1) * CN], wih, whh, bias,
                        ws1, wn1, b1.reshape(1, HID), wp,
                        b_pool.reshape(1, HID))
        hs.append(hk)
        qs.append(qk)
    q = jnp.concatenate(qs, axis=0)

    outs = []
    for k in range(NCHUNK):
        m2k = _sc_gather(q, idx_k[k].reshape(NW, per_w // c2, c2), HID,
                         jnp.float32).reshape(D, CN, HID)
        outs.append(_conv2(m2k, hs[k], ws2, wn2, b2.reshape(1, NCLS)))
    return jnp.concatenate(outs, axis=0)[:N]
